# Initial kernel scaffold; baseline (speedup 1.0000x reference)
import jax
import jax.numpy as jnp
from jax.experimental import pallas as pl


def kernel(z, rbf, idx_i, idx_j, emb, W, b):
    raise NotImplementedError("write your pallas kernel here")


# speedup vs baseline: 2.6517x; 2.6517x over previous
import functools

import jax
import jax.numpy as jnp
from jax import lax
from jax.experimental import pallas as pl
from jax.experimental.pallas import tpu as pltpu
from jax.experimental.pallas import tpu_sc as plsc

N = 10000
E = 320000
ATOM_F = 128
EDGE_F = 16
OUT_F = 128

BN = 1000
EB = 2000

_NC = 2
_NS = 16
_NW = _NC * _NS
_PER_W = E // _NW
_K = 80
_CHUNKS = _PER_W // _K


def _node_body(z_ref, emb_ref, w_ref, h_ref, gi_ref, gj_ref):
    zm1 = z_ref[...] - 1
    col = lax.broadcasted_iota(jnp.int32, (BN, ATOM_F), 1)
    onehot = (zm1 == col).astype(jnp.float32)
    h = jnp.dot(onehot, emb_ref[...], preferred_element_type=jnp.float32)
    h_ref[...] = h
    wi = w_ref[:, 0:ATOM_F]
    wj = w_ref[:, ATOM_F:2 * ATOM_F]
    dn = (((1,), (1,)), ((), ()))
    gi_ref[...] = lax.dot_general(h, wi, dn, preferred_element_type=jnp.float32)
    gj_ref[...] = lax.dot_general(h, wj, dn, preferred_element_type=jnp.float32)


def _node_call(z2d, emb_pad, w):
    return pl.pallas_call(
        _node_body,
        grid=(N // BN,),
        in_specs=[
            pl.BlockSpec((BN, 1), lambda i: (i, 0)),
            pl.BlockSpec((ATOM_F, ATOM_F), lambda i: (0, 0)),
            pl.BlockSpec((OUT_F, 2 * ATOM_F + EDGE_F), lambda i: (0, 0)),
        ],
        out_specs=[pl.BlockSpec((BN, ATOM_F), lambda i: (i, 0))] * 3,
        out_shape=[jax.ShapeDtypeStruct((N, ATOM_F), jnp.float32)] * 3,
    )(z2d, emb_pad, w)


@functools.lru_cache(maxsize=None)
def _make_edge_gather():
    mesh = plsc.VectorSubcoreMesh(core_axis_name="c", subcore_axis_name="s")

    @functools.partial(
        pl.kernel,
        mesh=mesh,
        out_type=jax.ShapeDtypeStruct((E, OUT_F), jnp.float32),
        scratch_types=[
            pltpu.VMEM((_K,), jnp.int32),
            pltpu.VMEM((_K,), jnp.int32),
            pltpu.VMEM((_K, OUT_F), jnp.float32),
            pltpu.VMEM((_K, OUT_F), jnp.float32),
            pltpu.SemaphoreType.DMA,
            pltpu.SemaphoreType.DMA,
        ],
    )
    def _edge_gather(gi_hbm, gj_hbm, ii_hbm, jj_hbm, out_hbm,
                     ii_v, jj_v, ri_v, rj_v, sem_i, sem_j):
        wid = lax.axis_index("s") * _NC + lax.axis_index("c")

        def chunk_body(c, carry):
            base = wid * _PER_W + c * _K
            pltpu.sync_copy(ii_hbm.at[pl.ds(base, _K)], ii_v)
            pltpu.sync_copy(jj_hbm.at[pl.ds(base, _K)], jj_v)
            cp_i = pltpu.async_copy(gi_hbm.at[ii_v], ri_v, sem_i)
            cp_j = pltpu.async_copy(gj_hbm.at[jj_v], rj_v, sem_j)
            cp_i.wait()
            cp_j.wait()

            def row_add(r, rcarry):
                for cb in range(OUT_F // 16):
                    sl = pl.ds(cb * 16, 16)
                    ri_v[r, sl] = ri_v[r, sl] + rj_v[r, sl]
                return rcarry

            lax.fori_loop(0, _K, row_add, 0)
            pltpu.sync_copy(ri_v, out_hbm.at[pl.ds(base, _K)])
            return carry

        lax.fori_loop(0, _CHUNKS, chunk_body, 0)

    return _edge_gather


def _edge_out_body(s_ref, rbf_ref, wr_ref, b_ref, o_ref):
    dn = (((1,), (1,)), ((), ()))
    t = lax.dot_general(rbf_ref[...], wr_ref[...], dn,
                        preferred_element_type=jnp.float32)
    x = s_ref[...] + t + b_ref[...]
    o_ref[...] = x / (1.0 + jnp.exp(-x))


def _edge_out_call(s, rbf, wr, b2d):
    return pl.pallas_call(
        _edge_out_body,
        grid=(E // EB,),
        in_specs=[
            pl.BlockSpec((EB, OUT_F), lambda i: (i, 0)),
            pl.BlockSpec((EB, EDGE_F), lambda i: (i, 0)),
            pl.BlockSpec((OUT_F, EDGE_F), lambda i: (0, 0)),
            pl.BlockSpec((1, OUT_F), lambda i: (0, 0)),
        ],
        out_specs=pl.BlockSpec((EB, OUT_F), lambda i: (i, 0)),
        out_shape=jax.ShapeDtypeStruct((E, OUT_F), jnp.float32),
    )(s, rbf, wr, b2d)


def kernel(z, rbf, idx_i, idx_j, emb, W, b):
    z2d = z.astype(jnp.int32).reshape(N, 1)
    emb_pad = jnp.pad(emb, ((0, ATOM_F - emb.shape[0]), (0, 0)))
    h, gi, gj = _node_call(z2d, emb_pad, W)
    s = _make_edge_gather()(gi, gj, idx_i.astype(jnp.int32),
                            idx_j.astype(jnp.int32))
    wr = lax.slice(W, (0, 2 * ATOM_F), (OUT_F, 2 * ATOM_F + EDGE_F))
    m_ij = _edge_out_call(s, rbf, wr, b.reshape(1, OUT_F))
    return (h, m_ij)
